# baseline (speedup 1.0000x reference)
import jax
import jax.numpy as jnp
from jax.experimental import pallas as pl
from jax.experimental.pallas import tpu as pltpu

DIM = 96
HEADS = 8
DH = DIM // HEADS
KV = 2 * DH
T = 197
G = HEADS * T


def _prep_body(x_ref, wq_ref, bq_ref, wk_ref, bk_ref, wv_ref, bv_ref,
               Wq_ref, Wk_ref, Wv_ref, q_ref, k_ref, v_ref):
    xv = x_ref[0]
    cls = xv[0:1, :]
    xs = xv[1:, :]
    xsr = xs.reshape(14, 14, 96)
    zr = jnp.zeros((1, 14, 96), jnp.float32)
    rows16 = jnp.concatenate([zr, xsr, zr], axis=0)
    zc = jnp.zeros((16, 1, 96), jnp.float32)
    p = jnp.concatenate([zc, rows16, zc], axis=1)

    def branch(w_ref, b_ref, W_ref, out_ref):
        acc = jnp.zeros((14, 14, 96), jnp.float32)
        for dy in range(3):
            for dx in range(3):
                tap = p[dy:dy + 14, dx:dx + 14, :]
                acc = acc + tap * w_ref[dy * 3 + dx][None, None, :]
        y = acc + b_ref[0][None, None, :]
        full = jnp.concatenate([cls, y.reshape(196, 96)], axis=0)
        out_ref[0] = jnp.dot(full, W_ref[:], preferred_element_type=jnp.float32)

    branch(wq_ref, bq_ref, Wq_ref, q_ref)
    branch(wk_ref, bk_ref, Wk_ref, k_ref)
    branch(wv_ref, bv_ref, Wv_ref, v_ref)


NT = 4


def _main_body(asg_ref, asgA_ref, kvg_ref, qp_ref, kp_ref, vp_ref, first_ref,
               rep_ref, oh0_ref, ohd_ref, oh0T_ref, ohdT_ref, cm8T_ref,
               cmc_ref, sel_ref, wc_ref, b2_ref, gall_ref, mall_ref,
               rsumT_ref, out_ref, tap_ref):
    @pl.when(jnp.logical_and(pl.program_id(0) == 0, pl.program_id(1) == 0))
    def _zero():
        tap_ref[...] = jnp.zeros((9, 8 * NT, 8, 192), jnp.float32)

    cm8T = cm8T_ref[:]
    cmc = cmc_ref[:]
    for i in range(NT):
        rows = asg_ref[0, 8 * i:8 * i + 8, :]
        rem2 = rows[:, 1:] * 2.0
        mp = jnp.max(rem2, axis=-1, keepdims=True)
        sp = jnp.sum(jnp.exp(rem2 - mp), axis=-1, keepdims=True)
        mn = jnp.max(-rem2, axis=-1, keepdims=True)
        sn = jnp.sum(jnp.exp(-rem2 - mn), axis=-1, keepdims=True)
        s4 = jnp.concatenate([mp, 1.0 / sp, mn, 1.0 / sn], axis=1)
        s4T = s4.T
        c0 = jnp.dot(s4T, oh0T_ref[:], preferred_element_type=jnp.float32)
        cd = jnp.dot(s4T, ohdT_ref[:], preferred_element_type=jnp.float32)
        mpA = c0[0:1, :] + cm8T * cd[0:1, :]
        ispA = c0[1:2, :] + cm8T * cd[1:2, :]
        mnA = c0[2:3, :] + cm8T * cd[2:3, :]
        isnA = c0[3:4, :] + cm8T * cd[3:4, :]
        x2 = asgA_ref[0, i] * 2.0
        posA = jnp.exp(x2 - mpA) * ispA
        negA = jnp.exp(-x2 - mnA) * isnA
        agsAT = 0.7 * posA + 0.3 - 0.3 * negA

        A192 = jax.lax.dot_general(
            agsAT, rep_ref[:], (((0,), (0,)), ((), ())),
            preferred_element_type=jnp.float32)
        kvg = kvg_ref[0, 8 * i:8 * i + 8, :]
        tk = jnp.dot(kvg, sel_ref[:], preferred_element_type=jnp.float32)
        t0 = jnp.dot(oh0_ref[:], tk, preferred_element_type=jnp.float32)
        td = jnp.dot(ohd_ref[:], tk, preferred_element_type=jnp.float32)
        f8 = A192 * (t0 + cmc * td)

        blocks = {}
        for eps, phi in ((0, 0), (0, 1), (1, 0), (1, 1)):
            st = (2 * eps + phi) * 56
            blocks[(eps, phi)] = f8[st:st + 56, :].reshape(7, 8, 192)
        for dy in range(3):
            eps, y0, ny = (1, 1, 6) if dy == 0 else \
                          (0, 0, 7) if dy == 1 else (1, 0, 7)
            for dx in range(3):
                phi, x0, nx = (1, 1, 6) if dx == 0 else \
                              (0, 0, 7) if dx == 1 else (1, 0, 7)
                tv = blocks[(eps, phi)][0:ny, 0:nx, :]
                tap_ref[dy * 3 + dx, 8 * i + y0:8 * i + y0 + ny,
                        x0:x0 + nx, :] = tv

    acc = jnp.zeros((64 * NT, 192), jnp.float32)
    for tapi in range(9):
        tap_all = tap_ref[tapi].reshape(64 * NT, 192)
        acc = acc + jnp.dot(tap_all, wc_ref[tapi],
                            preferred_element_type=jnp.float32)
    co_all = acc + b2_ref[0][None, :]

    coT_all = co_all.astype(jnp.bfloat16).T
    stack = jnp.concatenate(
        [coT_all[:, 64 * i:64 * i + 64] for i in range(NT)], axis=0)
    ybig = jnp.dot(stack, gall_ref[:],
                   preferred_element_type=jnp.float32)
    for i in range(NT):
        yi = ybig[192 * i:192 * i + 192, :] * mall_ref[:]
        zall = jnp.dot(rsumT_ref[:], yi,
                       preferred_element_type=jnp.float32)
        qs = qp_ref[0, i] * (96.0 ** -0.5)
        logits = jnp.zeros((8, 49), jnp.float32)
        for c in range(12):
            logits = logits + zall[:, c * 49:(c + 1) * 49] * qs[:, c:c + 1]
        fv = first_ref[0, i]
        kpr = kp_ref[0, i]
        vpr = vp_ref[0, i]
        logit0 = jnp.sum(qs * kpr, axis=1, keepdims=True) * fv
        m = jnp.maximum(jnp.max(logits, axis=1, keepdims=True), logit0)
        e = jnp.exp(logits - m)
        e0 = jnp.exp(logit0 - m)
        den = jnp.sum(e, axis=1, keepdims=True) + e0
        cols = [jnp.sum(e * zall[:, (12 + c) * 49:(13 + c) * 49], axis=1,
                        keepdims=True) for c in range(12)]
        o8 = (jnp.concatenate(cols, axis=1) + e0 * (vpr * fv)) / den
        out_ref[0, i] = o8


def _proj_body(x_ref, Wo_ref, bo_ref, out_ref):
    out_ref[...] = jnp.dot(x_ref[...], Wo_ref[...],
                           preferred_element_type=jnp.float32) + bo_ref[0][None, :]


@jax.jit
def _run(x, asg, wq_t, bq, wk_t, bk, wv_t, bv, Wq, Wk, Wv, Wc_t, b2, Wo, bo):
    B = x.shape[0]
    prep = pl.pallas_call(
        _prep_body,
        grid=(B,),
        in_specs=[
            pl.BlockSpec((1, T, DIM), lambda b: (b, 0, 0)),
            pl.BlockSpec((9, DIM), lambda b: (0, 0)),
            pl.BlockSpec((1, DIM), lambda b: (0, 0)),
            pl.BlockSpec((9, DIM), lambda b: (0, 0)),
            pl.BlockSpec((1, DIM), lambda b: (0, 0)),
            pl.BlockSpec((9, DIM), lambda b: (0, 0)),
            pl.BlockSpec((1, DIM), lambda b: (0, 0)),
            pl.BlockSpec((DIM, DIM), lambda b: (0, 0)),
            pl.BlockSpec((DIM, DIM), lambda b: (0, 0)),
            pl.BlockSpec((DIM, DIM), lambda b: (0, 0)),
        ],
        out_specs=[
            pl.BlockSpec((1, T, DIM), lambda b: (b, 0, 0)),
            pl.BlockSpec((1, T, DIM), lambda b: (b, 0, 0)),
            pl.BlockSpec((1, T, DIM), lambda b: (b, 0, 0)),
        ],
        out_shape=[jax.ShapeDtypeStruct((B, T, DIM), jnp.float32)] * 3,
    )
    qproj, kproj, vproj = prep(x, wq_t, bq, wk_t, bk, wv_t, bv, Wq, Wk, Wv)

    kh = kproj.reshape(B, T, HEADS, DH).transpose(0, 2, 1, 3)
    vh = vproj.reshape(B, T, HEADS, DH).transpose(0, 2, 1, 3)
    kv2g = jnp.concatenate([kh, vh], axis=-1).reshape(B, G, KV)
    asg2 = asg.reshape(B, G, T)
    asgA = asg2[:, :, 1:].reshape(B, T, 196, 8).transpose(0, 1, 3, 2)
    first_arr = asg[:, :, :, 0].transpose(0, 2, 1).reshape(B, T, HEADS, 1)
    qp4 = qproj.reshape(B, T, HEADS, DH)
    kp4 = kproj.reshape(B, T, HEADS, DH)
    vp4 = vproj.reshape(B, T, HEADS, DH)
    rep = jnp.repeat(jnp.eye(HEADS, dtype=jnp.float32), KV, axis=1)
    pp = jnp.arange(196)
    u0 = (8 * pp) // 196
    rho = (8 * pp) % 196
    oh0 = (jnp.arange(8)[None, :] == u0[:, None]).astype(jnp.float32)
    oh1 = (jnp.arange(8)[None, :] == jnp.minimum(u0 + 1, 7)[:, None]).astype(jnp.float32)
    ohd = oh1 - oh0
    cm8 = ((rho[:, None] + jnp.arange(8)[None, :]) >= 196).astype(jnp.float32)
    cmc = ((rho[:, None] + jnp.arange(2 * DIM)[None, :] // KV) >= 196).astype(jnp.float32)
    sel = ((jnp.arange(2 * DIM)[None, :] % KV) == jnp.arange(KV)[:, None]).astype(jnp.float32)
    pos = jnp.arange(224)
    kblk = pos // 56
    eps_, phi_ = kblk // 2, kblk % 2
    mrem = pos % 56
    alp, bet = mrem // 8, mrem % 8
    valid = (bet < 7).astype(jnp.float32)
    src = (2 * alp + eps_) * 14 + (2 * jnp.minimum(bet, 6) + phi_)
    asgA = asgA[:, :, :, src] * valid[None, None, None, :]
    oh0 = oh0[src, :] * valid[:, None]
    ohd = ohd[src, :] * valid[:, None]
    cm8 = cm8[src, :] * valid[:, None]
    cmc = cmc[src, :] * valid[:, None]
    cols = jnp.arange(24 * 49)
    mm = 24 * (cols % 49) + cols // 49
    r49 = mm % 49
    gall64 = (jnp.arange(64)[:, None] ==
              (8 * (r49 // 7) + r49 % 7)[None, :]).astype(jnp.float32)
    mall = ((jnp.arange(2 * DIM)[:, None] % KV) == (mm // 49)[None, :]).astype(jnp.float32)
    rsumT = ((jnp.arange(2 * DIM)[None, :] // KV) == jnp.arange(HEADS)[:, None]).astype(jnp.float32)

    NB = (T + NT - 1) // NT
    out8 = pl.pallas_call(
        _main_body,
        grid=(B, NB),
        in_specs=[
            pl.BlockSpec((1, 8 * NT, T), lambda b, n: (b, n, 0)),
            pl.BlockSpec((1, NT, HEADS, 224), lambda b, n: (b, n, 0, 0)),
            pl.BlockSpec((1, 8 * NT, KV), lambda b, n: (b, n, 0)),
            pl.BlockSpec((1, NT, HEADS, DH), lambda b, n: (b, n, 0, 0)),
            pl.BlockSpec((1, NT, HEADS, DH), lambda b, n: (b, n, 0, 0)),
            pl.BlockSpec((1, NT, HEADS, DH), lambda b, n: (b, n, 0, 0)),
            pl.BlockSpec((1, NT, HEADS, 1), lambda b, n: (b, n, 0, 0)),
            pl.BlockSpec((HEADS, 2 * DIM), lambda b, n: (0, 0)),
            pl.BlockSpec((224, HEADS), lambda b, n: (0, 0)),
            pl.BlockSpec((224, HEADS), lambda b, n: (0, 0)),
            pl.BlockSpec((HEADS, 224), lambda b, n: (0, 0)),
            pl.BlockSpec((HEADS, 224), lambda b, n: (0, 0)),
            pl.BlockSpec((HEADS, 224), lambda b, n: (0, 0)),
            pl.BlockSpec((224, 2 * DIM), lambda b, n: (0, 0)),
            pl.BlockSpec((KV, 2 * DIM), lambda b, n: (0, 0)),
            pl.BlockSpec((9, 2 * DIM, 2 * DIM), lambda b, n: (0, 0, 0)),
            pl.BlockSpec((1, 2 * DIM), lambda b, n: (0, 0)),
            pl.BlockSpec((64, 24 * 49), lambda b, n: (0, 0)),
            pl.BlockSpec((2 * DIM, 24 * 49), lambda b, n: (0, 0)),
            pl.BlockSpec((HEADS, 2 * DIM), lambda b, n: (0, 0)),
        ],
        out_specs=pl.BlockSpec((1, NT, HEADS, DH), lambda b, n: (b, n, 0, 0)),
        out_shape=jax.ShapeDtypeStruct((B, T, HEADS, DH), jnp.float32),
        scratch_shapes=[pltpu.VMEM((9, 8 * NT, 8, 192), jnp.float32)],
    )(asg2, asgA, kv2g, qp4, kp4, vp4, first_arr, rep, oh0, ohd,
      oh0.T, ohd.T, cm8.T, cmc, sel, Wc_t, b2,
      gall64.astype(jnp.bfloat16), mall, rsumT)

    o96 = out8.reshape(B * T, DIM)
    res = pl.pallas_call(
        _proj_body,
        grid=(1,),
        in_specs=[
            pl.BlockSpec((B * T, DIM), lambda i: (0, 0)),
            pl.BlockSpec((DIM, DIM), lambda i: (0, 0)),
            pl.BlockSpec((1, DIM), lambda i: (0, 0)),
        ],
        out_specs=pl.BlockSpec((B * T, DIM), lambda i: (0, 0)),
        out_shape=jax.ShapeDtypeStruct((B * T, DIM), jnp.float32),
    )(o96, Wo, bo)
    return res.reshape(B, T, DIM)


def kernel(x, h, w, attn_score_grad, conv_q_w, bn_q_g, bn_q_b, conv_k_w,
           bn_k_g, bn_k_b, conv_v_w, bn_v_g, bn_v_b, Wq, Wk, Wv, Cw, Cb,
           bn2_g, bn2_b, Wo, bo):
    eps = 1e-5
    sq = bn_q_g / jnp.sqrt(1.0 + eps)
    sk = bn_k_g / jnp.sqrt(1.0 + eps)
    sv = bn_v_g / jnp.sqrt(1.0 + eps)
    wq_t = (conv_q_w[:, 0] * sq[:, None, None]).transpose(1, 2, 0).reshape(9, DIM)
    wk_t = (conv_k_w[:, 0] * sk[:, None, None]).transpose(1, 2, 0).reshape(9, DIM)
    wv_t = (conv_v_w[:, 0] * sv[:, None, None]).transpose(1, 2, 0).reshape(9, DIM)
    s2 = bn2_g / jnp.sqrt(1.0 + eps)
    Wc_t = (Cw * s2[:, None, None, None]).transpose(2, 3, 1, 0).reshape(9, 2 * DIM, 2 * DIM)
    b2 = (Cb * s2 + bn2_b).reshape(1, 2 * DIM)
    return _run(x, attn_score_grad, wq_t, bn_q_b.reshape(1, DIM), wk_t,
                bn_k_b.reshape(1, DIM), wv_t, bn_v_b.reshape(1, DIM),
                Wq, Wk, Wv, Wc_t, b2, Wo, bo.reshape(1, DIM))


# speedup vs baseline: 1.1162x; 1.1162x over previous
import jax
import jax.numpy as jnp
from jax.experimental import pallas as pl
from jax.experimental.pallas import tpu as pltpu

DIM = 96
HEADS = 8
DH = DIM // HEADS
KV = 2 * DH
T = 197
G = HEADS * T


def _prep_body(x_ref, wq_ref, bq_ref, wk_ref, bk_ref, wv_ref, bv_ref,
               Wq_ref, Wk_ref, Wv_ref, q_ref, k_ref, v_ref):
    xv = x_ref[0]
    cls = xv[0:1, :]
    xs = xv[1:, :]
    xsr = xs.reshape(14, 14, 96)
    zr = jnp.zeros((1, 14, 96), jnp.float32)
    rows16 = jnp.concatenate([zr, xsr, zr], axis=0)
    zc = jnp.zeros((16, 1, 96), jnp.float32)
    p = jnp.concatenate([zc, rows16, zc], axis=1)

    def branch(w_ref, b_ref, W_ref, out_ref):
        acc = jnp.zeros((14, 14, 96), jnp.float32)
        for dy in range(3):
            for dx in range(3):
                tap = p[dy:dy + 14, dx:dx + 14, :]
                acc = acc + tap * w_ref[dy * 3 + dx][None, None, :]
        y = acc + b_ref[0][None, None, :]
        full = jnp.concatenate([cls, y.reshape(196, 96)], axis=0)
        out_ref[0] = jnp.dot(full, W_ref[:], preferred_element_type=jnp.float32)

    branch(wq_ref, bq_ref, Wq_ref, q_ref)
    branch(wk_ref, bk_ref, Wk_ref, k_ref)
    branch(wv_ref, bv_ref, Wv_ref, v_ref)


NT = 8


def _main_body(asg_ref, asgA_ref, kvg_ref, qp_ref, kp_ref, vp_ref, first_ref,
               rep_ref, oh0_ref, ohd_ref, oh0T_ref, ohdT_ref, cm8T_ref,
               cmc_ref, sel_ref, wc_ref, b2_ref, gall_ref, mall_ref,
               rsumT_ref, out_ref, tap_ref):
    @pl.when(jnp.logical_and(pl.program_id(0) == 0, pl.program_id(1) == 0))
    def _zero():
        tap_ref[...] = jnp.zeros((9, 8 * NT, 8, 192), jnp.float32)

    cm8T = cm8T_ref[:]
    cmc = cmc_ref[:]
    for i in range(NT):
        rows = asg_ref[0, 8 * i:8 * i + 8, :]
        rem2 = rows[:, 1:] * 2.0
        mp = jnp.max(rem2, axis=-1, keepdims=True)
        sp = jnp.sum(jnp.exp(rem2 - mp), axis=-1, keepdims=True)
        mn = jnp.max(-rem2, axis=-1, keepdims=True)
        sn = jnp.sum(jnp.exp(-rem2 - mn), axis=-1, keepdims=True)
        s4 = jnp.concatenate([mp, 1.0 / sp, mn, 1.0 / sn], axis=1)
        s4T = s4.T
        c0 = jnp.dot(s4T, oh0T_ref[:], preferred_element_type=jnp.float32)
        cd = jnp.dot(s4T, ohdT_ref[:], preferred_element_type=jnp.float32)
        mpA = c0[0:1, :] + cm8T * cd[0:1, :]
        ispA = c0[1:2, :] + cm8T * cd[1:2, :]
        mnA = c0[2:3, :] + cm8T * cd[2:3, :]
        isnA = c0[3:4, :] + cm8T * cd[3:4, :]
        x2 = asgA_ref[0, i] * 2.0
        posA = jnp.exp(x2 - mpA) * ispA
        negA = jnp.exp(-x2 - mnA) * isnA
        agsAT = 0.7 * posA + 0.3 - 0.3 * negA

        A192 = jax.lax.dot_general(
            agsAT, rep_ref[:], (((0,), (0,)), ((), ())),
            preferred_element_type=jnp.float32)
        kvg = kvg_ref[0, 8 * i:8 * i + 8, :]
        tk = jnp.dot(kvg, sel_ref[:], preferred_element_type=jnp.float32)
        t0 = jnp.dot(oh0_ref[:], tk, preferred_element_type=jnp.float32)
        td = jnp.dot(ohd_ref[:], tk, preferred_element_type=jnp.float32)
        f8 = A192 * (t0 + cmc * td)

        blocks = {}
        for eps, phi in ((0, 0), (0, 1), (1, 0), (1, 1)):
            st = (2 * eps + phi) * 56
            blocks[(eps, phi)] = f8[st:st + 56, :].reshape(7, 8, 192)
        for dy in range(3):
            eps, y0, ny = (1, 1, 6) if dy == 0 else \
                          (0, 0, 7) if dy == 1 else (1, 0, 7)
            for dx in range(3):
                phi, x0, nx = (1, 1, 6) if dx == 0 else \
                              (0, 0, 7) if dx == 1 else (1, 0, 7)
                tv = blocks[(eps, phi)][0:ny, 0:nx, :]
                tap_ref[dy * 3 + dx, 8 * i + y0:8 * i + y0 + ny,
                        x0:x0 + nx, :] = tv

    acc = jnp.zeros((64 * NT, 192), jnp.float32)
    for tapi in range(9):
        tap_all = tap_ref[tapi].reshape(64 * NT, 192)
        acc = acc + jnp.dot(tap_all, wc_ref[tapi],
                            preferred_element_type=jnp.float32)
    co_all = acc + b2_ref[0][None, :]

    coT_all = co_all.astype(jnp.bfloat16).T
    stack = jnp.concatenate(
        [coT_all[:, 64 * i:64 * i + 64] for i in range(NT)], axis=0)
    ybig = jnp.dot(stack, gall_ref[:],
                   preferred_element_type=jnp.float32)
    for i in range(NT):
        yi = ybig[192 * i:192 * i + 192, :] * mall_ref[:]
        zall = jnp.dot(rsumT_ref[:], yi,
                       preferred_element_type=jnp.float32)
        qs = qp_ref[0, i] * (96.0 ** -0.5)
        logits = jnp.zeros((8, 49), jnp.float32)
        for c in range(12):
            logits = logits + zall[:, c * 49:(c + 1) * 49] * qs[:, c:c + 1]
        fv = first_ref[0, i]
        kpr = kp_ref[0, i]
        vpr = vp_ref[0, i]
        logit0 = jnp.sum(qs * kpr, axis=1, keepdims=True) * fv
        m = jnp.maximum(jnp.max(logits, axis=1, keepdims=True), logit0)
        e = jnp.exp(logits - m)
        e0 = jnp.exp(logit0 - m)
        den = jnp.sum(e, axis=1, keepdims=True) + e0
        cols = [jnp.sum(e * zall[:, (12 + c) * 49:(13 + c) * 49], axis=1,
                        keepdims=True) for c in range(12)]
        o8 = (jnp.concatenate(cols, axis=1) + e0 * (vpr * fv)) / den
        out_ref[0, i] = o8


def _proj_body(x_ref, Wo_ref, bo_ref, out_ref):
    out_ref[...] = jnp.dot(x_ref[...], Wo_ref[...],
                           preferred_element_type=jnp.float32) + bo_ref[0][None, :]


@jax.jit
def _run(x, asg, wq_t, bq, wk_t, bk, wv_t, bv, Wq, Wk, Wv, Wc_t, b2, Wo, bo):
    B = x.shape[0]
    prep = pl.pallas_call(
        _prep_body,
        grid=(B,),
        in_specs=[
            pl.BlockSpec((1, T, DIM), lambda b: (b, 0, 0)),
            pl.BlockSpec((9, DIM), lambda b: (0, 0)),
            pl.BlockSpec((1, DIM), lambda b: (0, 0)),
            pl.BlockSpec((9, DIM), lambda b: (0, 0)),
            pl.BlockSpec((1, DIM), lambda b: (0, 0)),
            pl.BlockSpec((9, DIM), lambda b: (0, 0)),
            pl.BlockSpec((1, DIM), lambda b: (0, 0)),
            pl.BlockSpec((DIM, DIM), lambda b: (0, 0)),
            pl.BlockSpec((DIM, DIM), lambda b: (0, 0)),
            pl.BlockSpec((DIM, DIM), lambda b: (0, 0)),
        ],
        out_specs=[
            pl.BlockSpec((1, T, DIM), lambda b: (b, 0, 0)),
            pl.BlockSpec((1, T, DIM), lambda b: (b, 0, 0)),
            pl.BlockSpec((1, T, DIM), lambda b: (b, 0, 0)),
        ],
        out_shape=[jax.ShapeDtypeStruct((B, T, DIM), jnp.float32)] * 3,
    )
    qproj, kproj, vproj = prep(x, wq_t, bq, wk_t, bk, wv_t, bv, Wq, Wk, Wv)

    kh = kproj.reshape(B, T, HEADS, DH).transpose(0, 2, 1, 3)
    vh = vproj.reshape(B, T, HEADS, DH).transpose(0, 2, 1, 3)
    kv2g = jnp.concatenate([kh, vh], axis=-1).reshape(B, G, KV)
    asg2 = asg.reshape(B, G, T)
    asgA = asg2[:, :, 1:].reshape(B, T, 196, 8).transpose(0, 1, 3, 2)
    first_arr = asg[:, :, :, 0].transpose(0, 2, 1).reshape(B, T, HEADS, 1)
    qp4 = qproj.reshape(B, T, HEADS, DH)
    kp4 = kproj.reshape(B, T, HEADS, DH)
    vp4 = vproj.reshape(B, T, HEADS, DH)
    rep = jnp.repeat(jnp.eye(HEADS, dtype=jnp.float32), KV, axis=1)
    pp = jnp.arange(196)
    u0 = (8 * pp) // 196
    rho = (8 * pp) % 196
    oh0 = (jnp.arange(8)[None, :] == u0[:, None]).astype(jnp.float32)
    oh1 = (jnp.arange(8)[None, :] == jnp.minimum(u0 + 1, 7)[:, None]).astype(jnp.float32)
    ohd = oh1 - oh0
    cm8 = ((rho[:, None] + jnp.arange(8)[None, :]) >= 196).astype(jnp.float32)
    cmc = ((rho[:, None] + jnp.arange(2 * DIM)[None, :] // KV) >= 196).astype(jnp.float32)
    sel = ((jnp.arange(2 * DIM)[None, :] % KV) == jnp.arange(KV)[:, None]).astype(jnp.float32)
    pos = jnp.arange(224)
    kblk = pos // 56
    eps_, phi_ = kblk // 2, kblk % 2
    mrem = pos % 56
    alp, bet = mrem // 8, mrem % 8
    valid = (bet < 7).astype(jnp.float32)
    src = (2 * alp + eps_) * 14 + (2 * jnp.minimum(bet, 6) + phi_)
    asgA = asgA[:, :, :, src] * valid[None, None, None, :]
    oh0 = oh0[src, :] * valid[:, None]
    ohd = ohd[src, :] * valid[:, None]
    cm8 = cm8[src, :] * valid[:, None]
    cmc = cmc[src, :] * valid[:, None]
    cols = jnp.arange(24 * 49)
    mm = 24 * (cols % 49) + cols // 49
    r49 = mm % 49
    gall64 = (jnp.arange(64)[:, None] ==
              (8 * (r49 // 7) + r49 % 7)[None, :]).astype(jnp.float32)
    mall = ((jnp.arange(2 * DIM)[:, None] % KV) == (mm // 49)[None, :]).astype(jnp.float32)
    rsumT = ((jnp.arange(2 * DIM)[None, :] // KV) == jnp.arange(HEADS)[:, None]).astype(jnp.float32)

    NB = (T + NT - 1) // NT
    out8 = pl.pallas_call(
        _main_body,
        grid=(B, NB),
        in_specs=[
            pl.BlockSpec((1, 8 * NT, T), lambda b, n: (b, n, 0)),
            pl.BlockSpec((1, NT, HEADS, 224), lambda b, n: (b, n, 0, 0)),
            pl.BlockSpec((1, 8 * NT, KV), lambda b, n: (b, n, 0)),
            pl.BlockSpec((1, NT, HEADS, DH), lambda b, n: (b, n, 0, 0)),
            pl.BlockSpec((1, NT, HEADS, DH), lambda b, n: (b, n, 0, 0)),
            pl.BlockSpec((1, NT, HEADS, DH), lambda b, n: (b, n, 0, 0)),
            pl.BlockSpec((1, NT, HEADS, 1), lambda b, n: (b, n, 0, 0)),
            pl.BlockSpec((HEADS, 2 * DIM), lambda b, n: (0, 0)),
            pl.BlockSpec((224, HEADS), lambda b, n: (0, 0)),
            pl.BlockSpec((224, HEADS), lambda b, n: (0, 0)),
            pl.BlockSpec((HEADS, 224), lambda b, n: (0, 0)),
            pl.BlockSpec((HEADS, 224), lambda b, n: (0, 0)),
            pl.BlockSpec((HEADS, 224), lambda b, n: (0, 0)),
            pl.BlockSpec((224, 2 * DIM), lambda b, n: (0, 0)),
            pl.BlockSpec((KV, 2 * DIM), lambda b, n: (0, 0)),
            pl.BlockSpec((9, 2 * DIM, 2 * DIM), lambda b, n: (0, 0, 0)),
            pl.BlockSpec((1, 2 * DIM), lambda b, n: (0, 0)),
            pl.BlockSpec((64, 24 * 49), lambda b, n: (0, 0)),
            pl.BlockSpec((2 * DIM, 24 * 49), lambda b, n: (0, 0)),
            pl.BlockSpec((HEADS, 2 * DIM), lambda b, n: (0, 0)),
        ],
        out_specs=pl.BlockSpec((1, NT, HEADS, DH), lambda b, n: (b, n, 0, 0)),
        out_shape=jax.ShapeDtypeStruct((B, T, HEADS, DH), jnp.float32),
        scratch_shapes=[pltpu.VMEM((9, 8 * NT, 8, 192), jnp.float32)],
    )(asg2, asgA, kv2g, qp4, kp4, vp4, first_arr, rep, oh0, ohd,
      oh0.T, ohd.T, cm8.T, cmc, sel, Wc_t, b2,
      gall64.astype(jnp.bfloat16), mall, rsumT)

    o96 = out8.reshape(B * T, DIM)
    res = pl.pallas_call(
        _proj_body,
        grid=(1,),
        in_specs=[
            pl.BlockSpec((B * T, DIM), lambda i: (0, 0)),
            pl.BlockSpec((DIM, DIM), lambda i: (0, 0)),
            pl.BlockSpec((1, DIM), lambda i: (0, 0)),
        ],
        out_specs=pl.BlockSpec((B * T, DIM), lambda i: (0, 0)),
        out_shape=jax.ShapeDtypeStruct((B * T, DIM), jnp.float32),
    )(o96, Wo, bo)
    return res.reshape(B, T, DIM)


def kernel(x, h, w, attn_score_grad, conv_q_w, bn_q_g, bn_q_b, conv_k_w,
           bn_k_g, bn_k_b, conv_v_w, bn_v_g, bn_v_b, Wq, Wk, Wv, Cw, Cb,
           bn2_g, bn2_b, Wo, bo):
    eps = 1e-5
    sq = bn_q_g / jnp.sqrt(1.0 + eps)
    sk = bn_k_g / jnp.sqrt(1.0 + eps)
    sv = bn_v_g / jnp.sqrt(1.0 + eps)
    wq_t = (conv_q_w[:, 0] * sq[:, None, None]).transpose(1, 2, 0).reshape(9, DIM)
    wk_t = (conv_k_w[:, 0] * sk[:, None, None]).transpose(1, 2, 0).reshape(9, DIM)
    wv_t = (conv_v_w[:, 0] * sv[:, None, None]).transpose(1, 2, 0).reshape(9, DIM)
    s2 = bn2_g / jnp.sqrt(1.0 + eps)
    Wc_t = (Cw * s2[:, None, None, None]).transpose(2, 3, 1, 0).reshape(9, 2 * DIM, 2 * DIM)
    b2 = (Cb * s2 + bn2_b).reshape(1, 2 * DIM)
    return _run(x, attn_score_grad, wq_t, bn_q_b.reshape(1, DIM), wk_t,
                bn_k_b.reshape(1, DIM), wv_t, bn_v_b.reshape(1, DIM),
                Wq, Wk, Wv, Wc_t, b2, Wo, bo.reshape(1, DIM))


# speedup vs baseline: 1.1850x; 1.0616x over previous
import jax
import jax.numpy as jnp
from jax.experimental import pallas as pl
from jax.experimental.pallas import tpu as pltpu

DIM = 96
HEADS = 8
DH = DIM // HEADS
KV = 2 * DH
T = 197
G = HEADS * T


def _prep_body(x_ref, wq_ref, bq_ref, wk_ref, bk_ref, wv_ref, bv_ref,
               Wq_ref, Wk_ref, Wv_ref, q_ref, k_ref, v_ref):
    xv = x_ref[0]
    cls = xv[0:1, :]
    xs = xv[1:, :]
    xsr = xs.reshape(14, 14, 96)
    zr = jnp.zeros((1, 14, 96), jnp.float32)
    rows16 = jnp.concatenate([zr, xsr, zr], axis=0)
    zc = jnp.zeros((16, 1, 96), jnp.float32)
    p = jnp.concatenate([zc, rows16, zc], axis=1)

    def branch(w_ref, b_ref, W_ref, out_ref):
        acc = jnp.zeros((14, 14, 96), jnp.float32)
        for dy in range(3):
            for dx in range(3):
                tap = p[dy:dy + 14, dx:dx + 14, :]
                acc = acc + tap * w_ref[dy * 3 + dx][None, None, :]
        y = acc + b_ref[0][None, None, :]
        full = jnp.concatenate([cls, y.reshape(196, 96)], axis=0)
        out_ref[0] = jnp.dot(full, W_ref[:], preferred_element_type=jnp.float32)

    branch(wq_ref, bq_ref, Wq_ref, q_ref)
    branch(wk_ref, bk_ref, Wk_ref, k_ref)
    branch(wv_ref, bv_ref, Wv_ref, v_ref)


NT = 16


def _main_body(asg_ref, asgA_ref, kvg_ref, qp_ref, kp_ref, vp_ref, first_ref,
               rep_ref, oh0_ref, ohd_ref, oh0T_ref, ohdT_ref, cm8T_ref,
               cmc_ref, sel_ref, wc_ref, b2_ref, gall_ref, mall_ref,
               rsumT_ref, out_ref, tap_ref):
    @pl.when(jnp.logical_and(pl.program_id(0) == 0, pl.program_id(1) == 0))
    def _zero():
        tap_ref[...] = jnp.zeros((9, 8 * NT, 8, 192), jnp.float32)

    cm8T = cm8T_ref[:]
    cmc = cmc_ref[:]
    for i in range(NT):
        rows = asg_ref[0, 8 * i:8 * i + 8, :]
        rem2 = rows[:, 1:] * 2.0
        mp = jnp.max(rem2, axis=-1, keepdims=True)
        sp = jnp.sum(jnp.exp(rem2 - mp), axis=-1, keepdims=True)
        mn = jnp.max(-rem2, axis=-1, keepdims=True)
        sn = jnp.sum(jnp.exp(-rem2 - mn), axis=-1, keepdims=True)
        s4 = jnp.concatenate([mp, 1.0 / sp, mn, 1.0 / sn], axis=1)
        s4T = s4.T
        c0 = jnp.dot(s4T, oh0T_ref[:], preferred_element_type=jnp.float32)
        cd = jnp.dot(s4T, ohdT_ref[:], preferred_element_type=jnp.float32)
        mpA = c0[0:1, :] + cm8T * cd[0:1, :]
        ispA = c0[1:2, :] + cm8T * cd[1:2, :]
        mnA = c0[2:3, :] + cm8T * cd[2:3, :]
        isnA = c0[3:4, :] + cm8T * cd[3:4, :]
        x2 = asgA_ref[0, i] * 2.0
        posA = jnp.exp(x2 - mpA) * ispA
        negA = jnp.exp(-x2 - mnA) * isnA
        agsAT = 0.7 * posA + 0.3 - 0.3 * negA

        A192 = jax.lax.dot_general(
            agsAT, rep_ref[:], (((0,), (0,)), ((), ())),
            preferred_element_type=jnp.float32)
        kvg = kvg_ref[0, 8 * i:8 * i + 8, :]
        tk = jnp.dot(kvg, sel_ref[:], preferred_element_type=jnp.float32)
        t0 = jnp.dot(oh0_ref[:], tk, preferred_element_type=jnp.float32)
        td = jnp.dot(ohd_ref[:], tk, preferred_element_type=jnp.float32)
        f8 = A192 * (t0 + cmc * td)

        blocks = {}
        for eps, phi in ((0, 0), (0, 1), (1, 0), (1, 1)):
            st = (2 * eps + phi) * 56
            blocks[(eps, phi)] = f8[st:st + 56, :].reshape(7, 8, 192)
        for dy in range(3):
            eps, y0, ny = (1, 1, 6) if dy == 0 else \
                          (0, 0, 7) if dy == 1 else (1, 0, 7)
            for dx in range(3):
                phi, x0, nx = (1, 1, 6) if dx == 0 else \
                              (0, 0, 7) if dx == 1 else (1, 0, 7)
                tv = blocks[(eps, phi)][0:ny, 0:nx, :]
                tap_ref[dy * 3 + dx, 8 * i + y0:8 * i + y0 + ny,
                        x0:x0 + nx, :] = tv

    acc = jnp.zeros((64 * NT, 192), jnp.float32)
    for tapi in range(9):
        tap_all = tap_ref[tapi].reshape(64 * NT, 192)
        acc = acc + jnp.dot(tap_all, wc_ref[tapi],
                            preferred_element_type=jnp.float32)
    co_all = acc + b2_ref[0][None, :]

    coT_all = co_all.astype(jnp.bfloat16).T
    stack = jnp.concatenate(
        [coT_all[:, 64 * i:64 * i + 64] for i in range(NT)], axis=0)
    ybig = jnp.dot(stack, gall_ref[:],
                   preferred_element_type=jnp.float32)
    for i in range(NT):
        yi = ybig[192 * i:192 * i + 192, :] * mall_ref[:]
        zall = jnp.dot(rsumT_ref[:], yi,
                       preferred_element_type=jnp.float32)
        qs = qp_ref[0, i] * (96.0 ** -0.5)
        logits = jnp.zeros((8, 49), jnp.float32)
        for c in range(12):
            logits = logits + zall[:, c * 49:(c + 1) * 49] * qs[:, c:c + 1]
        fv = first_ref[0, i]
        kpr = kp_ref[0, i]
        vpr = vp_ref[0, i]
        logit0 = jnp.sum(qs * kpr, axis=1, keepdims=True) * fv
        m = jnp.maximum(jnp.max(logits, axis=1, keepdims=True), logit0)
        e = jnp.exp(logits - m)
        e0 = jnp.exp(logit0 - m)
        den = jnp.sum(e, axis=1, keepdims=True) + e0
        cols = [jnp.sum(e * zall[:, (12 + c) * 49:(13 + c) * 49], axis=1,
                        keepdims=True) for c in range(12)]
        o8 = (jnp.concatenate(cols, axis=1) + e0 * (vpr * fv)) / den
        out_ref[0, i] = o8


def _proj_body(x_ref, Wo_ref, bo_ref, out_ref):
    out_ref[...] = jnp.dot(x_ref[...], Wo_ref[...],
                           preferred_element_type=jnp.float32) + bo_ref[0][None, :]


@jax.jit
def _run(x, asg, wq_t, bq, wk_t, bk, wv_t, bv, Wq, Wk, Wv, Wc_t, b2, Wo, bo):
    B = x.shape[0]
    prep = pl.pallas_call(
        _prep_body,
        grid=(B,),
        in_specs=[
            pl.BlockSpec((1, T, DIM), lambda b: (b, 0, 0)),
            pl.BlockSpec((9, DIM), lambda b: (0, 0)),
            pl.BlockSpec((1, DIM), lambda b: (0, 0)),
            pl.BlockSpec((9, DIM), lambda b: (0, 0)),
            pl.BlockSpec((1, DIM), lambda b: (0, 0)),
            pl.BlockSpec((9, DIM), lambda b: (0, 0)),
            pl.BlockSpec((1, DIM), lambda b: (0, 0)),
            pl.BlockSpec((DIM, DIM), lambda b: (0, 0)),
            pl.BlockSpec((DIM, DIM), lambda b: (0, 0)),
            pl.BlockSpec((DIM, DIM), lambda b: (0, 0)),
        ],
        out_specs=[
            pl.BlockSpec((1, T, DIM), lambda b: (b, 0, 0)),
            pl.BlockSpec((1, T, DIM), lambda b: (b, 0, 0)),
            pl.BlockSpec((1, T, DIM), lambda b: (b, 0, 0)),
        ],
        out_shape=[jax.ShapeDtypeStruct((B, T, DIM), jnp.float32)] * 3,
    )
    qproj, kproj, vproj = prep(x, wq_t, bq, wk_t, bk, wv_t, bv, Wq, Wk, Wv)

    kh = kproj.reshape(B, T, HEADS, DH).transpose(0, 2, 1, 3)
    vh = vproj.reshape(B, T, HEADS, DH).transpose(0, 2, 1, 3)
    kv2g = jnp.concatenate([kh, vh], axis=-1).reshape(B, G, KV)
    asg2 = asg.reshape(B, G, T)
    asgA = asg2[:, :, 1:].reshape(B, T, 196, 8).transpose(0, 1, 3, 2)
    first_arr = asg[:, :, :, 0].transpose(0, 2, 1).reshape(B, T, HEADS, 1)
    qp4 = qproj.reshape(B, T, HEADS, DH)
    kp4 = kproj.reshape(B, T, HEADS, DH)
    vp4 = vproj.reshape(B, T, HEADS, DH)
    rep = jnp.repeat(jnp.eye(HEADS, dtype=jnp.float32), KV, axis=1)
    pp = jnp.arange(196)
    u0 = (8 * pp) // 196
    rho = (8 * pp) % 196
    oh0 = (jnp.arange(8)[None, :] == u0[:, None]).astype(jnp.float32)
    oh1 = (jnp.arange(8)[None, :] == jnp.minimum(u0 + 1, 7)[:, None]).astype(jnp.float32)
    ohd = oh1 - oh0
    cm8 = ((rho[:, None] + jnp.arange(8)[None, :]) >= 196).astype(jnp.float32)
    cmc = ((rho[:, None] + jnp.arange(2 * DIM)[None, :] // KV) >= 196).astype(jnp.float32)
    sel = ((jnp.arange(2 * DIM)[None, :] % KV) == jnp.arange(KV)[:, None]).astype(jnp.float32)
    pos = jnp.arange(224)
    kblk = pos // 56
    eps_, phi_ = kblk // 2, kblk % 2
    mrem = pos % 56
    alp, bet = mrem // 8, mrem % 8
    valid = (bet < 7).astype(jnp.float32)
    src = (2 * alp + eps_) * 14 + (2 * jnp.minimum(bet, 6) + phi_)
    asgA = asgA[:, :, :, src] * valid[None, None, None, :]
    oh0 = oh0[src, :] * valid[:, None]
    ohd = ohd[src, :] * valid[:, None]
    cm8 = cm8[src, :] * valid[:, None]
    cmc = cmc[src, :] * valid[:, None]
    cols = jnp.arange(24 * 49)
    mm = 24 * (cols % 49) + cols // 49
    r49 = mm % 49
    gall64 = (jnp.arange(64)[:, None] ==
              (8 * (r49 // 7) + r49 % 7)[None, :]).astype(jnp.float32)
    mall = ((jnp.arange(2 * DIM)[:, None] % KV) == (mm // 49)[None, :]).astype(jnp.float32)
    rsumT = ((jnp.arange(2 * DIM)[None, :] // KV) == jnp.arange(HEADS)[:, None]).astype(jnp.float32)

    NB = (T + NT - 1) // NT
    out8 = pl.pallas_call(
        _main_body,
        grid=(B, NB),
        in_specs=[
            pl.BlockSpec((1, 8 * NT, T), lambda b, n: (b, n, 0)),
            pl.BlockSpec((1, NT, HEADS, 224), lambda b, n: (b, n, 0, 0)),
            pl.BlockSpec((1, 8 * NT, KV), lambda b, n: (b, n, 0)),
            pl.BlockSpec((1, NT, HEADS, DH), lambda b, n: (b, n, 0, 0)),
            pl.BlockSpec((1, NT, HEADS, DH), lambda b, n: (b, n, 0, 0)),
            pl.BlockSpec((1, NT, HEADS, DH), lambda b, n: (b, n, 0, 0)),
            pl.BlockSpec((1, NT, HEADS, 1), lambda b, n: (b, n, 0, 0)),
            pl.BlockSpec((HEADS, 2 * DIM), lambda b, n: (0, 0)),
            pl.BlockSpec((224, HEADS), lambda b, n: (0, 0)),
            pl.BlockSpec((224, HEADS), lambda b, n: (0, 0)),
            pl.BlockSpec((HEADS, 224), lambda b, n: (0, 0)),
            pl.BlockSpec((HEADS, 224), lambda b, n: (0, 0)),
            pl.BlockSpec((HEADS, 224), lambda b, n: (0, 0)),
            pl.BlockSpec((224, 2 * DIM), lambda b, n: (0, 0)),
            pl.BlockSpec((KV, 2 * DIM), lambda b, n: (0, 0)),
            pl.BlockSpec((9, 2 * DIM, 2 * DIM), lambda b, n: (0, 0, 0)),
            pl.BlockSpec((1, 2 * DIM), lambda b, n: (0, 0)),
            pl.BlockSpec((64, 24 * 49), lambda b, n: (0, 0)),
            pl.BlockSpec((2 * DIM, 24 * 49), lambda b, n: (0, 0)),
            pl.BlockSpec((HEADS, 2 * DIM), lambda b, n: (0, 0)),
        ],
        out_specs=pl.BlockSpec((1, NT, HEADS, DH), lambda b, n: (b, n, 0, 0)),
        out_shape=jax.ShapeDtypeStruct((B, T, HEADS, DH), jnp.float32),
        scratch_shapes=[pltpu.VMEM((9, 8 * NT, 8, 192), jnp.float32)],
    )(asg2, asgA, kv2g, qp4, kp4, vp4, first_arr, rep, oh0, ohd,
      oh0.T, ohd.T, cm8.T, cmc, sel, Wc_t, b2,
      gall64.astype(jnp.bfloat16), mall, rsumT)

    o96 = out8.reshape(B * T, DIM)
    res = pl.pallas_call(
        _proj_body,
        grid=(1,),
        in_specs=[
            pl.BlockSpec((B * T, DIM), lambda i: (0, 0)),
            pl.BlockSpec((DIM, DIM), lambda i: (0, 0)),
            pl.BlockSpec((1, DIM), lambda i: (0, 0)),
        ],
        out_specs=pl.BlockSpec((B * T, DIM), lambda i: (0, 0)),
        out_shape=jax.ShapeDtypeStruct((B * T, DIM), jnp.float32),
    )(o96, Wo, bo)
    return res.reshape(B, T, DIM)


def kernel(x, h, w, attn_score_grad, conv_q_w, bn_q_g, bn_q_b, conv_k_w,
           bn_k_g, bn_k_b, conv_v_w, bn_v_g, bn_v_b, Wq, Wk, Wv, Cw, Cb,
           bn2_g, bn2_b, Wo, bo):
    eps = 1e-5
    sq = bn_q_g / jnp.sqrt(1.0 + eps)
    sk = bn_k_g / jnp.sqrt(1.0 + eps)
    sv = bn_v_g / jnp.sqrt(1.0 + eps)
    wq_t = (conv_q_w[:, 0] * sq[:, None, None]).transpose(1, 2, 0).reshape(9, DIM)
    wk_t = (conv_k_w[:, 0] * sk[:, None, None]).transpose(1, 2, 0).reshape(9, DIM)
    wv_t = (conv_v_w[:, 0] * sv[:, None, None]).transpose(1, 2, 0).reshape(9, DIM)
    s2 = bn2_g / jnp.sqrt(1.0 + eps)
    Wc_t = (Cw * s2[:, None, None, None]).transpose(2, 3, 1, 0).reshape(9, 2 * DIM, 2 * DIM)
    b2 = (Cb * s2 + bn2_b).reshape(1, 2 * DIM)
    return _run(x, attn_score_grad, wq_t, bn_q_b.reshape(1, DIM), wk_t,
                bn_k_b.reshape(1, DIM), wv_t, bn_v_b.reshape(1, DIM),
                Wq, Wk, Wv, Wc_t, b2, Wo, bo.reshape(1, DIM))


# speedup vs baseline: 1.1865x; 1.0012x over previous
import jax
import jax.numpy as jnp
from jax.experimental import pallas as pl
from jax.experimental.pallas import tpu as pltpu

DIM = 96
HEADS = 8
DH = DIM // HEADS
KV = 2 * DH
T = 197
G = HEADS * T


def _prep_body(x_ref, wq_ref, bq_ref, wk_ref, bk_ref, wv_ref, bv_ref,
               Wq_ref, Wk_ref, Wv_ref, q_ref, k_ref, v_ref):
    xv = x_ref[0]
    cls = xv[0:1, :]
    xs = xv[1:, :]
    xsr = xs.reshape(14, 14, 96)
    zr = jnp.zeros((1, 14, 96), jnp.float32)
    rows16 = jnp.concatenate([zr, xsr, zr], axis=0)
    zc = jnp.zeros((16, 1, 96), jnp.float32)
    p = jnp.concatenate([zc, rows16, zc], axis=1)

    def branch(w_ref, b_ref, W_ref, out_ref):
        acc = jnp.zeros((14, 14, 96), jnp.float32)
        for dy in range(3):
            for dx in range(3):
                tap = p[dy:dy + 14, dx:dx + 14, :]
                acc = acc + tap * w_ref[dy * 3 + dx][None, None, :]
        y = acc + b_ref[0][None, None, :]
        full = jnp.concatenate([cls, y.reshape(196, 96)], axis=0)
        out_ref[0] = jnp.dot(full, W_ref[:], preferred_element_type=jnp.float32)

    branch(wq_ref, bq_ref, Wq_ref, q_ref)
    branch(wk_ref, bk_ref, Wk_ref, k_ref)
    branch(wv_ref, bv_ref, Wv_ref, v_ref)


NT = 16


def _main_body(asg_ref, asgA_ref, kvg_ref, qp_ref, kp_ref, vp_ref, first_ref,
               rep_ref, oh0_ref, ohd_ref, oh0T_ref, ohdT_ref, cm8T_ref,
               cmc_ref, sel_ref, wc_ref, b2_ref, gall_ref, mall_ref,
               rsumT_ref, out_ref, tap_ref):
    @pl.when(jnp.logical_and(pl.program_id(0) == 0, pl.program_id(1) == 0))
    def _zero():
        tap_ref[...] = jnp.zeros((9, 8 * NT, 8, 192), jnp.bfloat16)

    cm8T = cm8T_ref[:]
    cmc = cmc_ref[:]
    for i in range(NT):
        rows = asg_ref[0, 8 * i:8 * i + 8, :]
        rem2 = rows[:, 1:] * 2.0
        mp = jnp.max(rem2, axis=-1, keepdims=True)
        sp = jnp.sum(jnp.exp(rem2 - mp), axis=-1, keepdims=True)
        mn = jnp.max(-rem2, axis=-1, keepdims=True)
        sn = jnp.sum(jnp.exp(-rem2 - mn), axis=-1, keepdims=True)
        s4 = jnp.concatenate([mp, 1.0 / sp, mn, 1.0 / sn], axis=1)
        s4T = s4.T
        c0 = jnp.dot(s4T, oh0T_ref[:], preferred_element_type=jnp.float32)
        cd = jnp.dot(s4T, ohdT_ref[:], preferred_element_type=jnp.float32)
        mpA = c0[0:1, :] + cm8T * cd[0:1, :]
        ispA = c0[1:2, :] + cm8T * cd[1:2, :]
        mnA = c0[2:3, :] + cm8T * cd[2:3, :]
        isnA = c0[3:4, :] + cm8T * cd[3:4, :]
        x2 = asgA_ref[0, i] * 2.0
        posA = jnp.exp(x2 - mpA) * ispA
        negA = jnp.exp(-x2 - mnA) * isnA
        agsAT = 0.7 * posA + 0.3 - 0.3 * negA

        A192 = jax.lax.dot_general(
            agsAT, rep_ref[:], (((0,), (0,)), ((), ())),
            preferred_element_type=jnp.float32)
        kvg = kvg_ref[0, 8 * i:8 * i + 8, :]
        tk = jnp.dot(kvg, sel_ref[:], preferred_element_type=jnp.float32)
        t0 = jnp.dot(oh0_ref[:], tk, preferred_element_type=jnp.float32)
        td = jnp.dot(ohd_ref[:], tk, preferred_element_type=jnp.float32)
        f8 = (A192 * (t0 + cmc * td)).astype(jnp.bfloat16)

        blocks = {}
        for eps, phi in ((0, 0), (0, 1), (1, 0), (1, 1)):
            st = (2 * eps + phi) * 56
            blocks[(eps, phi)] = f8[st:st + 56, :].reshape(7, 8, 192)
        for dy in range(3):
            eps, y0, ny = (1, 1, 6) if dy == 0 else \
                          (0, 0, 7) if dy == 1 else (1, 0, 7)
            for dx in range(3):
                phi, x0, nx = (1, 1, 6) if dx == 0 else \
                              (0, 0, 7) if dx == 1 else (1, 0, 7)
                tv = blocks[(eps, phi)][0:ny, 0:nx, :]
                tap_ref[dy * 3 + dx, 8 * i + y0:8 * i + y0 + ny,
                        x0:x0 + nx, :] = tv

    acc = jnp.zeros((64 * NT, 192), jnp.float32)
    for tapi in range(9):
        tap_all = tap_ref[tapi].reshape(64 * NT, 192)
        acc = acc + jnp.dot(tap_all, wc_ref[tapi],
                            preferred_element_type=jnp.float32)
    co_all = acc + b2_ref[0][None, :]

    coT_all = co_all.astype(jnp.bfloat16).T
    stack = jnp.concatenate(
        [coT_all[:, 64 * i:64 * i + 64] for i in range(NT)], axis=0)
    ybig = jnp.dot(stack, gall_ref[:],
                   preferred_element_type=jnp.float32)
    for i in range(NT):
        yi = ybig[192 * i:192 * i + 192, :] * mall_ref[:]
        zall = jnp.dot(rsumT_ref[:], yi,
                       preferred_element_type=jnp.float32)
        qs = qp_ref[0, i] * (96.0 ** -0.5)
        logits = jnp.zeros((8, 49), jnp.float32)
        for c in range(12):
            logits = logits + zall[:, c * 49:(c + 1) * 49] * qs[:, c:c + 1]
        fv = first_ref[0, i]
        kpr = kp_ref[0, i]
        vpr = vp_ref[0, i]
        logit0 = jnp.sum(qs * kpr, axis=1, keepdims=True) * fv
        m = jnp.maximum(jnp.max(logits, axis=1, keepdims=True), logit0)
        e = jnp.exp(logits - m)
        e0 = jnp.exp(logit0 - m)
        den = jnp.sum(e, axis=1, keepdims=True) + e0
        cols = [jnp.sum(e * zall[:, (12 + c) * 49:(13 + c) * 49], axis=1,
                        keepdims=True) for c in range(12)]
        o8 = (jnp.concatenate(cols, axis=1) + e0 * (vpr * fv)) / den
        out_ref[0, i] = o8


def _proj_body(x_ref, Wo_ref, bo_ref, out_ref):
    out_ref[...] = jnp.dot(x_ref[...], Wo_ref[...],
                           preferred_element_type=jnp.float32) + bo_ref[0][None, :]


@jax.jit
def _run(x, asg, wq_t, bq, wk_t, bk, wv_t, bv, Wq, Wk, Wv, Wc_t, b2, Wo, bo):
    B = x.shape[0]
    prep = pl.pallas_call(
        _prep_body,
        grid=(B,),
        in_specs=[
            pl.BlockSpec((1, T, DIM), lambda b: (b, 0, 0)),
            pl.BlockSpec((9, DIM), lambda b: (0, 0)),
            pl.BlockSpec((1, DIM), lambda b: (0, 0)),
            pl.BlockSpec((9, DIM), lambda b: (0, 0)),
            pl.BlockSpec((1, DIM), lambda b: (0, 0)),
            pl.BlockSpec((9, DIM), lambda b: (0, 0)),
            pl.BlockSpec((1, DIM), lambda b: (0, 0)),
            pl.BlockSpec((DIM, DIM), lambda b: (0, 0)),
            pl.BlockSpec((DIM, DIM), lambda b: (0, 0)),
            pl.BlockSpec((DIM, DIM), lambda b: (0, 0)),
        ],
        out_specs=[
            pl.BlockSpec((1, T, DIM), lambda b: (b, 0, 0)),
            pl.BlockSpec((1, T, DIM), lambda b: (b, 0, 0)),
            pl.BlockSpec((1, T, DIM), lambda b: (b, 0, 0)),
        ],
        out_shape=[jax.ShapeDtypeStruct((B, T, DIM), jnp.float32)] * 3,
    )
    qproj, kproj, vproj = prep(x, wq_t, bq, wk_t, bk, wv_t, bv, Wq, Wk, Wv)

    kh = kproj.reshape(B, T, HEADS, DH).transpose(0, 2, 1, 3)
    vh = vproj.reshape(B, T, HEADS, DH).transpose(0, 2, 1, 3)
    kv2g = jnp.concatenate([kh, vh], axis=-1).reshape(B, G, KV)
    asg2 = asg.reshape(B, G, T)
    asgA = asg2[:, :, 1:].reshape(B, T, 196, 8).transpose(0, 1, 3, 2)
    first_arr = asg[:, :, :, 0].transpose(0, 2, 1).reshape(B, T, HEADS, 1)
    qp4 = qproj.reshape(B, T, HEADS, DH)
    kp4 = kproj.reshape(B, T, HEADS, DH)
    vp4 = vproj.reshape(B, T, HEADS, DH)
    rep = jnp.repeat(jnp.eye(HEADS, dtype=jnp.float32), KV, axis=1)
    pp = jnp.arange(196)
    u0 = (8 * pp) // 196
    rho = (8 * pp) % 196
    oh0 = (jnp.arange(8)[None, :] == u0[:, None]).astype(jnp.float32)
    oh1 = (jnp.arange(8)[None, :] == jnp.minimum(u0 + 1, 7)[:, None]).astype(jnp.float32)
    ohd = oh1 - oh0
    cm8 = ((rho[:, None] + jnp.arange(8)[None, :]) >= 196).astype(jnp.float32)
    cmc = ((rho[:, None] + jnp.arange(2 * DIM)[None, :] // KV) >= 196).astype(jnp.float32)
    sel = ((jnp.arange(2 * DIM)[None, :] % KV) == jnp.arange(KV)[:, None]).astype(jnp.float32)
    pos = jnp.arange(224)
    kblk = pos // 56
    eps_, phi_ = kblk // 2, kblk % 2
    mrem = pos % 56
    alp, bet = mrem // 8, mrem % 8
    valid = (bet < 7).astype(jnp.float32)
    src = (2 * alp + eps_) * 14 + (2 * jnp.minimum(bet, 6) + phi_)
    asgA = asgA[:, :, :, src] * valid[None, None, None, :]
    oh0 = oh0[src, :] * valid[:, None]
    ohd = ohd[src, :] * valid[:, None]
    cm8 = cm8[src, :] * valid[:, None]
    cmc = cmc[src, :] * valid[:, None]
    cols = jnp.arange(24 * 49)
    mm = 24 * (cols % 49) + cols // 49
    r49 = mm % 49
    gall64 = (jnp.arange(64)[:, None] ==
              (8 * (r49 // 7) + r49 % 7)[None, :]).astype(jnp.float32)
    mall = ((jnp.arange(2 * DIM)[:, None] % KV) == (mm // 49)[None, :]).astype(jnp.float32)
    rsumT = ((jnp.arange(2 * DIM)[None, :] // KV) == jnp.arange(HEADS)[:, None]).astype(jnp.float32)

    NB = (T + NT - 1) // NT
    out8 = pl.pallas_call(
        _main_body,
        grid=(B, NB),
        in_specs=[
            pl.BlockSpec((1, 8 * NT, T), lambda b, n: (b, n, 0)),
            pl.BlockSpec((1, NT, HEADS, 224), lambda b, n: (b, n, 0, 0)),
            pl.BlockSpec((1, 8 * NT, KV), lambda b, n: (b, n, 0)),
            pl.BlockSpec((1, NT, HEADS, DH), lambda b, n: (b, n, 0, 0)),
            pl.BlockSpec((1, NT, HEADS, DH), lambda b, n: (b, n, 0, 0)),
            pl.BlockSpec((1, NT, HEADS, DH), lambda b, n: (b, n, 0, 0)),
            pl.BlockSpec((1, NT, HEADS, 1), lambda b, n: (b, n, 0, 0)),
            pl.BlockSpec((HEADS, 2 * DIM), lambda b, n: (0, 0)),
            pl.BlockSpec((224, HEADS), lambda b, n: (0, 0)),
            pl.BlockSpec((224, HEADS), lambda b, n: (0, 0)),
            pl.BlockSpec((HEADS, 224), lambda b, n: (0, 0)),
            pl.BlockSpec((HEADS, 224), lambda b, n: (0, 0)),
            pl.BlockSpec((HEADS, 224), lambda b, n: (0, 0)),
            pl.BlockSpec((224, 2 * DIM), lambda b, n: (0, 0)),
            pl.BlockSpec((KV, 2 * DIM), lambda b, n: (0, 0)),
            pl.BlockSpec((9, 2 * DIM, 2 * DIM), lambda b, n: (0, 0, 0)),
            pl.BlockSpec((1, 2 * DIM), lambda b, n: (0, 0)),
            pl.BlockSpec((64, 24 * 49), lambda b, n: (0, 0)),
            pl.BlockSpec((2 * DIM, 24 * 49), lambda b, n: (0, 0)),
            pl.BlockSpec((HEADS, 2 * DIM), lambda b, n: (0, 0)),
        ],
        out_specs=pl.BlockSpec((1, NT, HEADS, DH), lambda b, n: (b, n, 0, 0)),
        out_shape=jax.ShapeDtypeStruct((B, T, HEADS, DH), jnp.float32),
        scratch_shapes=[pltpu.VMEM((9, 8 * NT, 8, 192), jnp.bfloat16)],
    )(asg2, asgA, kv2g, qp4, kp4, vp4, first_arr, rep, oh0, ohd,
      oh0.T, ohd.T, cm8.T, cmc, sel, Wc_t.astype(jnp.bfloat16), b2,
      gall64.astype(jnp.bfloat16), mall, rsumT)

    o96 = out8.reshape(B * T, DIM)
    res = pl.pallas_call(
        _proj_body,
        grid=(1,),
        in_specs=[
            pl.BlockSpec((B * T, DIM), lambda i: (0, 0)),
            pl.BlockSpec((DIM, DIM), lambda i: (0, 0)),
            pl.BlockSpec((1, DIM), lambda i: (0, 0)),
        ],
        out_specs=pl.BlockSpec((B * T, DIM), lambda i: (0, 0)),
        out_shape=jax.ShapeDtypeStruct((B * T, DIM), jnp.float32),
    )(o96, Wo, bo)
    return res.reshape(B, T, DIM)


def kernel(x, h, w, attn_score_grad, conv_q_w, bn_q_g, bn_q_b, conv_k_w,
           bn_k_g, bn_k_b, conv_v_w, bn_v_g, bn_v_b, Wq, Wk, Wv, Cw, Cb,
           bn2_g, bn2_b, Wo, bo):
    eps = 1e-5
    sq = bn_q_g / jnp.sqrt(1.0 + eps)
    sk = bn_k_g / jnp.sqrt(1.0 + eps)
    sv = bn_v_g / jnp.sqrt(1.0 + eps)
    wq_t = (conv_q_w[:, 0] * sq[:, None, None]).transpose(1, 2, 0).reshape(9, DIM)
    wk_t = (conv_k_w[:, 0] * sk[:, None, None]).transpose(1, 2, 0).reshape(9, DIM)
    wv_t = (conv_v_w[:, 0] * sv[:, None, None]).transpose(1, 2, 0).reshape(9, DIM)
    s2 = bn2_g / jnp.sqrt(1.0 + eps)
    Wc_t = (Cw * s2[:, None, None, None]).transpose(2, 3, 1, 0).reshape(9, 2 * DIM, 2 * DIM)
    b2 = (Cb * s2 + bn2_b).reshape(1, 2 * DIM)
    return _run(x, attn_score_grad, wq_t, bn_q_b.reshape(1, DIM), wk_t,
                bn_k_b.reshape(1, DIM), wv_t, bn_v_b.reshape(1, DIM),
                Wq, Wk, Wv, Wc_t, b2, Wo, bo.reshape(1, DIM))


# speedup vs baseline: 1.1883x; 1.0016x over previous
import jax
import jax.numpy as jnp
from jax.experimental import pallas as pl
from jax.experimental.pallas import tpu as pltpu

DIM = 96
HEADS = 8
DH = DIM // HEADS
KV = 2 * DH
T = 197
G = HEADS * T


def _prep_body(x_ref, wq_ref, bq_ref, wk_ref, bk_ref, wv_ref, bv_ref,
               Wq_ref, Wk_ref, Wv_ref, q_ref, k_ref, v_ref, kv_ref):
    xv = x_ref[0]
    cls = xv[0:1, :]
    xs = xv[1:, :]
    xsr = xs.reshape(14, 14, 96)
    zr = jnp.zeros((1, 14, 96), jnp.float32)
    rows16 = jnp.concatenate([zr, xsr, zr], axis=0)
    zc = jnp.zeros((16, 1, 96), jnp.float32)
    p = jnp.concatenate([zc, rows16, zc], axis=1)

    def branch(w_ref, b_ref, W_ref, out_ref):
        acc = jnp.zeros((14, 14, 96), jnp.float32)
        for dy in range(3):
            for dx in range(3):
                tap = p[dy:dy + 14, dx:dx + 14, :]
                acc = acc + tap * w_ref[dy * 3 + dx][None, None, :]
        y = acc + b_ref[0][None, None, :]
        full = jnp.concatenate([cls, y.reshape(196, 96)], axis=0)
        proj = jnp.dot(full, W_ref[:], preferred_element_type=jnp.float32)
        out_ref[0] = proj
        return proj

    branch(wq_ref, bq_ref, Wq_ref, q_ref)
    kproj = branch(wk_ref, bk_ref, Wk_ref, k_ref)
    vproj = branch(wv_ref, bv_ref, Wv_ref, v_ref)
    for h in range(HEADS):
        kv_ref[0, 197 * h:197 * (h + 1), :] = jnp.concatenate(
            [kproj[:, DH * h:DH * (h + 1)], vproj[:, DH * h:DH * (h + 1)]],
            axis=1)


NT = 16


def _main_body(asg_ref, asgA_ref, kvg_ref, qp_ref, kp_ref, vp_ref, first_ref,
               rep_ref, oh0_ref, ohd_ref, oh0T_ref, ohdT_ref, cm8T_ref,
               cmc_ref, sel_ref, wc_ref, b2_ref, gall_ref, mall_ref,
               rsumT_ref, out_ref, tap_ref):
    @pl.when(jnp.logical_and(pl.program_id(0) == 0, pl.program_id(1) == 0))
    def _zero():
        tap_ref[...] = jnp.zeros((9, 8 * NT, 8, 192), jnp.float32)

    cm8T = cm8T_ref[:]
    cmc = cmc_ref[:]
    for i in range(NT):
        rows = asg_ref[0, 8 * i:8 * i + 8, :]
        rem2 = rows[:, 1:] * 2.0
        mp = jnp.max(rem2, axis=-1, keepdims=True)
        sp = jnp.sum(jnp.exp(rem2 - mp), axis=-1, keepdims=True)
        mn = jnp.max(-rem2, axis=-1, keepdims=True)
        sn = jnp.sum(jnp.exp(-rem2 - mn), axis=-1, keepdims=True)
        s4 = jnp.concatenate([mp, 1.0 / sp, mn, 1.0 / sn], axis=1)
        s4T = s4.T
        c0 = jnp.dot(s4T, oh0T_ref[:], preferred_element_type=jnp.float32)
        cd = jnp.dot(s4T, ohdT_ref[:], preferred_element_type=jnp.float32)
        mpA = c0[0:1, :] + cm8T * cd[0:1, :]
        ispA = c0[1:2, :] + cm8T * cd[1:2, :]
        mnA = c0[2:3, :] + cm8T * cd[2:3, :]
        isnA = c0[3:4, :] + cm8T * cd[3:4, :]
        x2 = asgA_ref[0, i] * 2.0
        posA = jnp.exp(x2 - mpA) * ispA
        negA = jnp.exp(-x2 - mnA) * isnA
        agsAT = 0.7 * posA + 0.3 - 0.3 * negA

        A192 = jax.lax.dot_general(
            agsAT, rep_ref[:], (((0,), (0,)), ((), ())),
            preferred_element_type=jnp.float32)
        kvg = kvg_ref[0, 8 * i:8 * i + 8, :]
        tk = jnp.dot(kvg, sel_ref[:], preferred_element_type=jnp.float32)
        t0 = jnp.dot(oh0_ref[:], tk, preferred_element_type=jnp.float32)
        td = jnp.dot(ohd_ref[:], tk, preferred_element_type=jnp.float32)
        f8 = A192 * (t0 + cmc * td)

        blocks = {}
        for eps, phi in ((0, 0), (0, 1), (1, 0), (1, 1)):
            st = (2 * eps + phi) * 56
            blocks[(eps, phi)] = f8[st:st + 56, :].reshape(7, 8, 192)
        for dy in range(3):
            eps, y0, ny = (1, 1, 6) if dy == 0 else \
                          (0, 0, 7) if dy == 1 else (1, 0, 7)
            for dx in range(3):
                phi, x0, nx = (1, 1, 6) if dx == 0 else \
                              (0, 0, 7) if dx == 1 else (1, 0, 7)
                tv = blocks[(eps, phi)][0:ny, 0:nx, :]
                tap_ref[dy * 3 + dx, 8 * i + y0:8 * i + y0 + ny,
                        x0:x0 + nx, :] = tv

    acc = jnp.zeros((64 * NT, 192), jnp.float32)
    for tapi in range(9):
        tap_all = tap_ref[tapi].reshape(64 * NT, 192)
        acc = acc + jnp.dot(tap_all, wc_ref[tapi],
                            preferred_element_type=jnp.float32)
    co_all = acc + b2_ref[0][None, :]

    coT_all = co_all.astype(jnp.bfloat16).T
    stack = jnp.concatenate(
        [coT_all[:, 64 * i:64 * i + 64] for i in range(NT)], axis=0)
    ybig = jnp.dot(stack, gall_ref[:],
                   preferred_element_type=jnp.float32)
    for i in range(NT):
        yi = ybig[192 * i:192 * i + 192, :] * mall_ref[:]
        zall = jnp.dot(rsumT_ref[:], yi,
                       preferred_element_type=jnp.float32)
        qs = qp_ref[0, i] * (96.0 ** -0.5)
        logits = jnp.zeros((8, 49), jnp.float32)
        for c in range(12):
            logits = logits + zall[:, c * 49:(c + 1) * 49] * qs[:, c:c + 1]
        fv = first_ref[0, i]
        kpr = kp_ref[0, i]
        vpr = vp_ref[0, i]
        logit0 = jnp.sum(qs * kpr, axis=1, keepdims=True) * fv
        m = jnp.maximum(jnp.max(logits, axis=1, keepdims=True), logit0)
        e = jnp.exp(logits - m)
        e0 = jnp.exp(logit0 - m)
        den = jnp.sum(e, axis=1, keepdims=True) + e0
        cols = [jnp.sum(e * zall[:, (12 + c) * 49:(13 + c) * 49], axis=1,
                        keepdims=True) for c in range(12)]
        o8 = (jnp.concatenate(cols, axis=1) + e0 * (vpr * fv)) / den
        out_ref[0, i] = o8


def _proj_body(x_ref, Wo_ref, bo_ref, out_ref):
    out_ref[...] = jnp.dot(x_ref[...], Wo_ref[...],
                           preferred_element_type=jnp.float32) + bo_ref[0][None, :]


@jax.jit
def _run(x, asg, wq_t, bq, wk_t, bk, wv_t, bv, Wq, Wk, Wv, Wc_t, b2, Wo, bo):
    B = x.shape[0]
    prep = pl.pallas_call(
        _prep_body,
        grid=(B,),
        in_specs=[
            pl.BlockSpec((1, T, DIM), lambda b: (b, 0, 0)),
            pl.BlockSpec((9, DIM), lambda b: (0, 0)),
            pl.BlockSpec((1, DIM), lambda b: (0, 0)),
            pl.BlockSpec((9, DIM), lambda b: (0, 0)),
            pl.BlockSpec((1, DIM), lambda b: (0, 0)),
            pl.BlockSpec((9, DIM), lambda b: (0, 0)),
            pl.BlockSpec((1, DIM), lambda b: (0, 0)),
            pl.BlockSpec((DIM, DIM), lambda b: (0, 0)),
            pl.BlockSpec((DIM, DIM), lambda b: (0, 0)),
            pl.BlockSpec((DIM, DIM), lambda b: (0, 0)),
        ],
        out_specs=[
            pl.BlockSpec((1, T, DIM), lambda b: (b, 0, 0)),
            pl.BlockSpec((1, T, DIM), lambda b: (b, 0, 0)),
            pl.BlockSpec((1, T, DIM), lambda b: (b, 0, 0)),
            pl.BlockSpec((1, G, KV), lambda b: (b, 0, 0)),
        ],
        out_shape=[jax.ShapeDtypeStruct((B, T, DIM), jnp.float32)] * 3
        + [jax.ShapeDtypeStruct((B, G, KV), jnp.float32)],
    )
    qproj, kproj, vproj, kv2g = prep(x, wq_t, bq, wk_t, bk, wv_t, bv,
                                     Wq, Wk, Wv)

    asg2 = asg.reshape(B, G, T)
    asgA = asg2[:, :, 1:].reshape(B, T, 196, 8).transpose(0, 1, 3, 2)
    first_arr = asg[:, :, :, 0].transpose(0, 2, 1).reshape(B, T, HEADS, 1)
    qp4 = qproj.reshape(B, T, HEADS, DH)
    kp4 = kproj.reshape(B, T, HEADS, DH)
    vp4 = vproj.reshape(B, T, HEADS, DH)
    rep = jnp.repeat(jnp.eye(HEADS, dtype=jnp.float32), KV, axis=1)
    pp = jnp.arange(196)
    u0 = (8 * pp) // 196
    rho = (8 * pp) % 196
    oh0 = (jnp.arange(8)[None, :] == u0[:, None]).astype(jnp.float32)
    oh1 = (jnp.arange(8)[None, :] == jnp.minimum(u0 + 1, 7)[:, None]).astype(jnp.float32)
    ohd = oh1 - oh0
    cm8 = ((rho[:, None] + jnp.arange(8)[None, :]) >= 196).astype(jnp.float32)
    cmc = ((rho[:, None] + jnp.arange(2 * DIM)[None, :] // KV) >= 196).astype(jnp.float32)
    sel = ((jnp.arange(2 * DIM)[None, :] % KV) == jnp.arange(KV)[:, None]).astype(jnp.float32)
    pos = jnp.arange(224)
    kblk = pos // 56
    eps_, phi_ = kblk // 2, kblk % 2
    mrem = pos % 56
    alp, bet = mrem // 8, mrem % 8
    valid = (bet < 7).astype(jnp.float32)
    src = (2 * alp + eps_) * 14 + (2 * jnp.minimum(bet, 6) + phi_)
    asgA = asgA[:, :, :, src] * valid[None, None, None, :]
    oh0 = oh0[src, :] * valid[:, None]
    ohd = ohd[src, :] * valid[:, None]
    cm8 = cm8[src, :] * valid[:, None]
    cmc = cmc[src, :] * valid[:, None]
    cols = jnp.arange(24 * 49)
    mm = 24 * (cols % 49) + cols // 49
    r49 = mm % 49
    gall64 = (jnp.arange(64)[:, None] ==
              (8 * (r49 // 7) + r49 % 7)[None, :]).astype(jnp.float32)
    mall = ((jnp.arange(2 * DIM)[:, None] % KV) == (mm // 49)[None, :]).astype(jnp.float32)
    rsumT = ((jnp.arange(2 * DIM)[None, :] // KV) == jnp.arange(HEADS)[:, None]).astype(jnp.float32)

    NB = (T + NT - 1) // NT
    out8 = pl.pallas_call(
        _main_body,
        grid=(B, NB),
        in_specs=[
            pl.BlockSpec((1, 8 * NT, T), lambda b, n: (b, n, 0)),
            pl.BlockSpec((1, NT, HEADS, 224), lambda b, n: (b, n, 0, 0)),
            pl.BlockSpec((1, 8 * NT, KV), lambda b, n: (b, n, 0)),
            pl.BlockSpec((1, NT, HEADS, DH), lambda b, n: (b, n, 0, 0)),
            pl.BlockSpec((1, NT, HEADS, DH), lambda b, n: (b, n, 0, 0)),
            pl.BlockSpec((1, NT, HEADS, DH), lambda b, n: (b, n, 0, 0)),
            pl.BlockSpec((1, NT, HEADS, 1), lambda b, n: (b, n, 0, 0)),
            pl.BlockSpec((HEADS, 2 * DIM), lambda b, n: (0, 0)),
            pl.BlockSpec((224, HEADS), lambda b, n: (0, 0)),
            pl.BlockSpec((224, HEADS), lambda b, n: (0, 0)),
            pl.BlockSpec((HEADS, 224), lambda b, n: (0, 0)),
            pl.BlockSpec((HEADS, 224), lambda b, n: (0, 0)),
            pl.BlockSpec((HEADS, 224), lambda b, n: (0, 0)),
            pl.BlockSpec((224, 2 * DIM), lambda b, n: (0, 0)),
            pl.BlockSpec((KV, 2 * DIM), lambda b, n: (0, 0)),
            pl.BlockSpec((9, 2 * DIM, 2 * DIM), lambda b, n: (0, 0, 0)),
            pl.BlockSpec((1, 2 * DIM), lambda b, n: (0, 0)),
            pl.BlockSpec((64, 24 * 49), lambda b, n: (0, 0)),
            pl.BlockSpec((2 * DIM, 24 * 49), lambda b, n: (0, 0)),
            pl.BlockSpec((HEADS, 2 * DIM), lambda b, n: (0, 0)),
        ],
        out_specs=pl.BlockSpec((1, NT, HEADS, DH), lambda b, n: (b, n, 0, 0)),
        out_shape=jax.ShapeDtypeStruct((B, T, HEADS, DH), jnp.float32),
        scratch_shapes=[pltpu.VMEM((9, 8 * NT, 8, 192), jnp.float32)],
    )(asg2, asgA, kv2g, qp4, kp4, vp4, first_arr, rep, oh0, ohd,
      oh0.T, ohd.T, cm8.T, cmc, sel, Wc_t, b2,
      gall64.astype(jnp.bfloat16), mall, rsumT)

    o96 = out8.reshape(B * T, DIM)
    res = pl.pallas_call(
        _proj_body,
        grid=(1,),
        in_specs=[
            pl.BlockSpec((B * T, DIM), lambda i: (0, 0)),
            pl.BlockSpec((DIM, DIM), lambda i: (0, 0)),
            pl.BlockSpec((1, DIM), lambda i: (0, 0)),
        ],
        out_specs=pl.BlockSpec((B * T, DIM), lambda i: (0, 0)),
        out_shape=jax.ShapeDtypeStruct((B * T, DIM), jnp.float32),
    )(o96, Wo, bo)
    return res.reshape(B, T, DIM)


def kernel(x, h, w, attn_score_grad, conv_q_w, bn_q_g, bn_q_b, conv_k_w,
           bn_k_g, bn_k_b, conv_v_w, bn_v_g, bn_v_b, Wq, Wk, Wv, Cw, Cb,
           bn2_g, bn2_b, Wo, bo):
    eps = 1e-5
    sq = bn_q_g / jnp.sqrt(1.0 + eps)
    sk = bn_k_g / jnp.sqrt(1.0 + eps)
    sv = bn_v_g / jnp.sqrt(1.0 + eps)
    wq_t = (conv_q_w[:, 0] * sq[:, None, None]).transpose(1, 2, 0).reshape(9, DIM)
    wk_t = (conv_k_w[:, 0] * sk[:, None, None]).transpose(1, 2, 0).reshape(9, DIM)
    wv_t = (conv_v_w[:, 0] * sv[:, None, None]).transpose(1, 2, 0).reshape(9, DIM)
    s2 = bn2_g / jnp.sqrt(1.0 + eps)
    Wc_t = (Cw * s2[:, None, None, None]).transpose(2, 3, 1, 0).reshape(9, 2 * DIM, 2 * DIM)
    b2 = (Cb * s2 + bn2_b).reshape(1, 2 * DIM)
    return _run(x, attn_score_grad, wq_t, bn_q_b.reshape(1, DIM), wk_t,
                bn_k_b.reshape(1, DIM), wv_t, bn_v_b.reshape(1, DIM),
                Wq, Wk, Wv, Wc_t, b2, Wo, bo.reshape(1, DIM))


# speedup vs baseline: 1.2199x; 1.0266x over previous
import jax
import jax.numpy as jnp
from jax.experimental import pallas as pl
from jax.experimental.pallas import tpu as pltpu

DIM = 96
HEADS = 8
DH = DIM // HEADS
KV = 2 * DH
T = 197
G = HEADS * T


def _prep_body(x_ref, wq_ref, bq_ref, wk_ref, bk_ref, wv_ref, bv_ref,
               Wq_ref, Wk_ref, Wv_ref, q_ref, k_ref, v_ref, kv_ref):
    xv = x_ref[0]
    cls = xv[0:1, :]
    xs = xv[1:, :]
    xsr = xs.reshape(14, 14, 96)
    zr = jnp.zeros((1, 14, 96), jnp.float32)
    rows16 = jnp.concatenate([zr, xsr, zr], axis=0)
    zc = jnp.zeros((16, 1, 96), jnp.float32)
    p = jnp.concatenate([zc, rows16, zc], axis=1)

    def branch(w_ref, b_ref, W_ref, out_ref):
        acc = jnp.zeros((14, 14, 96), jnp.float32)
        for dy in range(3):
            for dx in range(3):
                tap = p[dy:dy + 14, dx:dx + 14, :]
                acc = acc + tap * w_ref[dy * 3 + dx][None, None, :]
        y = acc + b_ref[0][None, None, :]
        full = jnp.concatenate([cls, y.reshape(196, 96)], axis=0)
        proj = jnp.dot(full, W_ref[:], preferred_element_type=jnp.float32)
        out_ref[0] = proj
        return proj

    branch(wq_ref, bq_ref, Wq_ref, q_ref)
    kproj = branch(wk_ref, bk_ref, Wk_ref, k_ref)
    vproj = branch(wv_ref, bv_ref, Wv_ref, v_ref)
    for h in range(HEADS):
        kv_ref[0, 197 * h:197 * (h + 1), :] = jnp.concatenate(
            [kproj[:, DH * h:DH * (h + 1)], vproj[:, DH * h:DH * (h + 1)]],
            axis=1)


NT = 16


def _main_body(asg_ref, asgA_ref, kvg_ref, qp_ref, kp_ref, vp_ref, first_ref,
               rep_ref, oh0_ref, ohd_ref, oh0T_ref, ohdT_ref, cm8T_ref,
               cmc_ref, sel_ref, wc_ref, b2_ref, gall_ref, mall_ref,
               rsumT_ref, out_ref, tap_ref):
    @pl.when(jnp.logical_and(pl.program_id(0) == 0, pl.program_id(1) == 0))
    def _zero():
        tap_ref[...] = jnp.zeros((9, 7 * NT, 8, 192), jnp.float32)

    cm8T = cm8T_ref[:]
    cmc = cmc_ref[:]
    for i in range(NT):
        rows = asg_ref[0, 8 * i:8 * i + 8, :]
        rem2 = rows[:, 1:] * 2.0
        mp = jnp.max(rem2, axis=-1, keepdims=True)
        sp = jnp.sum(jnp.exp(rem2 - mp), axis=-1, keepdims=True)
        mn = jnp.max(-rem2, axis=-1, keepdims=True)
        sn = jnp.sum(jnp.exp(-rem2 - mn), axis=-1, keepdims=True)
        s4 = jnp.concatenate([mp, 1.0 / sp, mn, 1.0 / sn], axis=1)
        s4T = s4.T
        c0 = jnp.dot(s4T, oh0T_ref[:], preferred_element_type=jnp.float32)
        cd = jnp.dot(s4T, ohdT_ref[:], preferred_element_type=jnp.float32)
        mpA = c0[0:1, :] + cm8T * cd[0:1, :]
        ispA = c0[1:2, :] + cm8T * cd[1:2, :]
        mnA = c0[2:3, :] + cm8T * cd[2:3, :]
        isnA = c0[3:4, :] + cm8T * cd[3:4, :]
        x2 = asgA_ref[0, i] * 2.0
        posA = jnp.exp(x2 - mpA) * ispA
        negA = jnp.exp(-x2 - mnA) * isnA
        agsAT = 0.7 * posA + 0.3 - 0.3 * negA

        A192 = jax.lax.dot_general(
            agsAT, rep_ref[:], (((0,), (0,)), ((), ())),
            preferred_element_type=jnp.float32)
        kvg = kvg_ref[0, 8 * i:8 * i + 8, :]
        tk = jnp.dot(kvg, sel_ref[:], preferred_element_type=jnp.float32)
        t0 = jnp.dot(oh0_ref[:], tk, preferred_element_type=jnp.float32)
        td = jnp.dot(ohd_ref[:], tk, preferred_element_type=jnp.float32)
        f8 = A192 * (t0 + cmc * td)

        blocks = {}
        for eps, phi in ((0, 0), (0, 1), (1, 0), (1, 1)):
            st = (2 * eps + phi) * 56
            blocks[(eps, phi)] = f8[st:st + 56, :].reshape(7, 8, 192)
        for dy in range(3):
            eps, y0, ny = (1, 1, 6) if dy == 0 else \
                          (0, 0, 7) if dy == 1 else (1, 0, 7)
            for dx in range(3):
                phi, x0, nx = (1, 1, 6) if dx == 0 else \
                              (0, 0, 7) if dx == 1 else (1, 0, 7)
                tv = blocks[(eps, phi)][0:ny, 0:nx, :]
                tap_ref[dy * 3 + dx, 7 * i + y0:7 * i + y0 + ny,
                        x0:x0 + nx, :] = tv

    acc = jnp.zeros((56 * NT, 192), jnp.float32)
    for tapi in range(9):
        tap_all = tap_ref[tapi].reshape(56 * NT, 192)
        acc = acc + jnp.dot(tap_all, wc_ref[tapi],
                            preferred_element_type=jnp.float32)
    co_all = acc + b2_ref[0][None, :]

    coT_all = co_all.astype(jnp.bfloat16).T
    stack = jnp.concatenate(
        [coT_all[:, 56 * i:56 * i + 56] for i in range(NT)], axis=0)
    ybig = jnp.dot(stack, gall_ref[:],
                   preferred_element_type=jnp.float32)
    for i in range(NT):
        yi = ybig[192 * i:192 * i + 192, :] * mall_ref[:]
        zall = jnp.dot(rsumT_ref[:], yi,
                       preferred_element_type=jnp.float32)
        qs = qp_ref[0, i] * (96.0 ** -0.5)
        logits = jnp.zeros((8, 49), jnp.float32)
        for c in range(12):
            logits = logits + zall[:, c * 49:(c + 1) * 49] * qs[:, c:c + 1]
        fv = first_ref[0, i]
        kpr = kp_ref[0, i]
        vpr = vp_ref[0, i]
        logit0 = jnp.sum(qs * kpr, axis=1, keepdims=True) * fv
        m = jnp.maximum(jnp.max(logits, axis=1, keepdims=True), logit0)
        e = jnp.exp(logits - m)
        e0 = jnp.exp(logit0 - m)
        den = jnp.sum(e, axis=1, keepdims=True) + e0
        cols = [jnp.sum(e * zall[:, (12 + c) * 49:(13 + c) * 49], axis=1,
                        keepdims=True) for c in range(12)]
        o8 = (jnp.concatenate(cols, axis=1) + e0 * (vpr * fv)) / den
        out_ref[0, i] = o8


def _proj_body(x_ref, Wo_ref, bo_ref, out_ref):
    out_ref[...] = jnp.dot(x_ref[...], Wo_ref[...],
                           preferred_element_type=jnp.float32) + bo_ref[0][None, :]


@jax.jit
def _run(x, asg, wq_t, bq, wk_t, bk, wv_t, bv, Wq, Wk, Wv, Wc_t, b2, Wo, bo):
    B = x.shape[0]
    prep = pl.pallas_call(
        _prep_body,
        grid=(B,),
        in_specs=[
            pl.BlockSpec((1, T, DIM), lambda b: (b, 0, 0)),
            pl.BlockSpec((9, DIM), lambda b: (0, 0)),
            pl.BlockSpec((1, DIM), lambda b: (0, 0)),
            pl.BlockSpec((9, DIM), lambda b: (0, 0)),
            pl.BlockSpec((1, DIM), lambda b: (0, 0)),
            pl.BlockSpec((9, DIM), lambda b: (0, 0)),
            pl.BlockSpec((1, DIM), lambda b: (0, 0)),
            pl.BlockSpec((DIM, DIM), lambda b: (0, 0)),
            pl.BlockSpec((DIM, DIM), lambda b: (0, 0)),
            pl.BlockSpec((DIM, DIM), lambda b: (0, 0)),
        ],
        out_specs=[
            pl.BlockSpec((1, T, DIM), lambda b: (b, 0, 0)),
            pl.BlockSpec((1, T, DIM), lambda b: (b, 0, 0)),
            pl.BlockSpec((1, T, DIM), lambda b: (b, 0, 0)),
            pl.BlockSpec((1, G, KV), lambda b: (b, 0, 0)),
        ],
        out_shape=[jax.ShapeDtypeStruct((B, T, DIM), jnp.float32)] * 3
        + [jax.ShapeDtypeStruct((B, G, KV), jnp.float32)],
    )
    qproj, kproj, vproj, kv2g = prep(x, wq_t, bq, wk_t, bk, wv_t, bv,
                                     Wq, Wk, Wv)

    asg2 = asg.reshape(B, G, T)
    asgA = asg2[:, :, 1:].reshape(B, T, 196, 8).transpose(0, 1, 3, 2)
    first_arr = asg[:, :, :, 0].transpose(0, 2, 1).reshape(B, T, HEADS, 1)
    qp4 = qproj.reshape(B, T, HEADS, DH)
    kp4 = kproj.reshape(B, T, HEADS, DH)
    vp4 = vproj.reshape(B, T, HEADS, DH)
    rep = jnp.repeat(jnp.eye(HEADS, dtype=jnp.float32), KV, axis=1)
    pp = jnp.arange(196)
    u0 = (8 * pp) // 196
    rho = (8 * pp) % 196
    oh0 = (jnp.arange(8)[None, :] == u0[:, None]).astype(jnp.float32)
    oh1 = (jnp.arange(8)[None, :] == jnp.minimum(u0 + 1, 7)[:, None]).astype(jnp.float32)
    ohd = oh1 - oh0
    cm8 = ((rho[:, None] + jnp.arange(8)[None, :]) >= 196).astype(jnp.float32)
    cmc = ((rho[:, None] + jnp.arange(2 * DIM)[None, :] // KV) >= 196).astype(jnp.float32)
    sel = ((jnp.arange(2 * DIM)[None, :] % KV) == jnp.arange(KV)[:, None]).astype(jnp.float32)
    pos = jnp.arange(224)
    kblk = pos // 56
    eps_, phi_ = kblk // 2, kblk % 2
    mrem = pos % 56
    alp, bet = mrem // 8, mrem % 8
    valid = (bet < 7).astype(jnp.float32)
    src = (2 * alp + eps_) * 14 + (2 * jnp.minimum(bet, 6) + phi_)
    asgA = asgA[:, :, :, src] * valid[None, None, None, :]
    oh0 = oh0[src, :] * valid[:, None]
    ohd = ohd[src, :] * valid[:, None]
    cm8 = cm8[src, :] * valid[:, None]
    cmc = cmc[src, :] * valid[:, None]
    cols = jnp.arange(24 * 49)
    mm = 24 * (cols % 49) + cols // 49
    r49 = mm % 49
    gall64 = (jnp.arange(56)[:, None] ==
              (8 * (r49 // 7) + r49 % 7)[None, :]).astype(jnp.float32)
    mall = ((jnp.arange(2 * DIM)[:, None] % KV) == (mm // 49)[None, :]).astype(jnp.float32)
    rsumT = ((jnp.arange(2 * DIM)[None, :] // KV) == jnp.arange(HEADS)[:, None]).astype(jnp.float32)

    NB = (T + NT - 1) // NT
    out8 = pl.pallas_call(
        _main_body,
        grid=(B, NB),
        in_specs=[
            pl.BlockSpec((1, 8 * NT, T), lambda b, n: (b, n, 0)),
            pl.BlockSpec((1, NT, HEADS, 224), lambda b, n: (b, n, 0, 0)),
            pl.BlockSpec((1, 8 * NT, KV), lambda b, n: (b, n, 0)),
            pl.BlockSpec((1, NT, HEADS, DH), lambda b, n: (b, n, 0, 0)),
            pl.BlockSpec((1, NT, HEADS, DH), lambda b, n: (b, n, 0, 0)),
            pl.BlockSpec((1, NT, HEADS, DH), lambda b, n: (b, n, 0, 0)),
            pl.BlockSpec((1, NT, HEADS, 1), lambda b, n: (b, n, 0, 0)),
            pl.BlockSpec((HEADS, 2 * DIM), lambda b, n: (0, 0)),
            pl.BlockSpec((224, HEADS), lambda b, n: (0, 0)),
            pl.BlockSpec((224, HEADS), lambda b, n: (0, 0)),
            pl.BlockSpec((HEADS, 224), lambda b, n: (0, 0)),
            pl.BlockSpec((HEADS, 224), lambda b, n: (0, 0)),
            pl.BlockSpec((HEADS, 224), lambda b, n: (0, 0)),
            pl.BlockSpec((224, 2 * DIM), lambda b, n: (0, 0)),
            pl.BlockSpec((KV, 2 * DIM), lambda b, n: (0, 0)),
            pl.BlockSpec((9, 2 * DIM, 2 * DIM), lambda b, n: (0, 0, 0)),
            pl.BlockSpec((1, 2 * DIM), lambda b, n: (0, 0)),
            pl.BlockSpec((56, 24 * 49), lambda b, n: (0, 0)),
            pl.BlockSpec((2 * DIM, 24 * 49), lambda b, n: (0, 0)),
            pl.BlockSpec((HEADS, 2 * DIM), lambda b, n: (0, 0)),
        ],
        out_specs=pl.BlockSpec((1, NT, HEADS, DH), lambda b, n: (b, n, 0, 0)),
        out_shape=jax.ShapeDtypeStruct((B, T, HEADS, DH), jnp.float32),
        scratch_shapes=[pltpu.VMEM((9, 7 * NT, 8, 192), jnp.float32)],
    )(asg2, asgA, kv2g, qp4, kp4, vp4, first_arr, rep, oh0, ohd,
      oh0.T, ohd.T, cm8.T, cmc, sel, Wc_t, b2,
      gall64.astype(jnp.bfloat16), mall, rsumT)

    o96 = out8.reshape(B * T, DIM)
    res = pl.pallas_call(
        _proj_body,
        grid=(1,),
        in_specs=[
            pl.BlockSpec((B * T, DIM), lambda i: (0, 0)),
            pl.BlockSpec((DIM, DIM), lambda i: (0, 0)),
            pl.BlockSpec((1, DIM), lambda i: (0, 0)),
        ],
        out_specs=pl.BlockSpec((B * T, DIM), lambda i: (0, 0)),
        out_shape=jax.ShapeDtypeStruct((B * T, DIM), jnp.float32),
    )(o96, Wo, bo)
    return res.reshape(B, T, DIM)


def kernel(x, h, w, attn_score_grad, conv_q_w, bn_q_g, bn_q_b, conv_k_w,
           bn_k_g, bn_k_b, conv_v_w, bn_v_g, bn_v_b, Wq, Wk, Wv, Cw, Cb,
           bn2_g, bn2_b, Wo, bo):
    eps = 1e-5
    sq = bn_q_g / jnp.sqrt(1.0 + eps)
    sk = bn_k_g / jnp.sqrt(1.0 + eps)
    sv = bn_v_g / jnp.sqrt(1.0 + eps)
    wq_t = (conv_q_w[:, 0] * sq[:, None, None]).transpose(1, 2, 0).reshape(9, DIM)
    wk_t = (conv_k_w[:, 0] * sk[:, None, None]).transpose(1, 2, 0).reshape(9, DIM)
    wv_t = (conv_v_w[:, 0] * sv[:, None, None]).transpose(1, 2, 0).reshape(9, DIM)
    s2 = bn2_g / jnp.sqrt(1.0 + eps)
    Wc_t = (Cw * s2[:, None, None, None]).transpose(2, 3, 1, 0).reshape(9, 2 * DIM, 2 * DIM)
    b2 = (Cb * s2 + bn2_b).reshape(1, 2 * DIM)
    return _run(x, attn_score_grad, wq_t, bn_q_b.reshape(1, DIM), wk_t,
                bn_k_b.reshape(1, DIM), wv_t, bn_v_b.reshape(1, DIM),
                Wq, Wk, Wv, Wc_t, b2, Wo, bo.reshape(1, DIM))
